# baseline (speedup 1.0000x reference)
import functools

import jax
import jax.numpy as jnp
from jax import lax
from jax.experimental import pallas as pl
from jax.experimental.pallas import tpu as pltpu
from jax.experimental.pallas import tpu_sc as plsc

N = 10000
D = 128
E = 320000

NC = 2
NS = 16
NW = NC * NS
CHUNK = 128
B = 16
NBLK = 5
K = NBLK * B
PER_W = K * CHUNK
E_PAD = PER_W * NW
STRIPE = 632
STRIPE_LAST = N - STRIPE * (NS - 1)
N_PAD = N + 8

ROW_BLK = 1000


def _sc_aggregate(x, rows_r, cols_r):
    mesh = plsc.VectorSubcoreMesh(
        core_axis_name="c", subcore_axis_name="s",
        num_cores=NC, num_subcores=NS)

    @functools.partial(
        pl.kernel,
        out_type=jax.ShapeDtypeStruct((NC, N, D), jnp.float32),
        mesh=mesh,
        scratch_types=[
            pltpu.VMEM((2, B, CHUNK), jnp.int32),
            pltpu.VMEM((2, B, CHUNK), jnp.int32),
            pltpu.VMEM((CHUNK, D), jnp.float32),
            pltpu.VMEM((CHUNK, D), jnp.float32),
            pltpu.VMEM_SHARED((N_PAD, D), jnp.float32),
            pltpu.SemaphoreType.DMA,
            pltpu.SemaphoreType.DMA,
            pltpu.SemaphoreType.DMA,
            pltpu.SemaphoreType.DMA,
        ],
    )
    def agg_kernel(x_hbm, rows_hbm, cols_hbm, out_hbm,
                   row_blk, col_blk, buf0, buf1, agg_sh,
                   sem0, sem1, semr, semc):
        c = lax.axis_index("c")
        s = lax.axis_index("s")
        pltpu.sync_copy(rows_hbm.at[c, s, 0], row_blk.at[0])
        pltpu.sync_copy(cols_hbm.at[c, s, 0], col_blk.at[0])
        pltpu.async_copy(rows_hbm.at[c, s, 1], row_blk.at[1], semr)
        pltpu.async_copy(cols_hbm.at[c, s, 1], col_blk.at[1], semc)
        off = pl.multiple_of(s * STRIPE, 8)

        @pl.when(s < NS - 1)
        def _():
            pltpu.sync_copy(x_hbm.at[pl.ds(off, STRIPE)],
                            agg_sh.at[pl.ds(off, STRIPE)])

        @pl.when(s == NS - 1)
        def _():
            pltpu.sync_copy(x_hbm.at[pl.ds((NS - 1) * STRIPE, STRIPE_LAST)],
                            agg_sh.at[pl.ds((NS - 1) * STRIPE, STRIPE_LAST)])

        plsc.subcore_barrier()

        pltpu.async_copy(x_hbm.at[col_blk.at[0, 0]], buf0, sem0)
        pltpu.async_copy(x_hbm.at[col_blk.at[0, 1]], buf1, sem1)

        @pl.loop(0, K, step=2)
        def _(j):
            blk = j // B
            r = j - blk * B
            slot = blk % 2
            slot2 = 1 - slot
            for b, (buf, sem) in enumerate(((buf0, sem0), (buf1, sem1))):
                pltpu.make_async_copy(
                    x_hbm.at[pl.ds(0, CHUNK)], buf, sem).wait()
                pltpu.sync_copy(buf, agg_sh.at[row_blk.at[slot, r + b]],
                                add=True)

                @pl.when(r < B - 2)
                def _(buf=buf, sem=sem, b=b):
                    pltpu.async_copy(
                        x_hbm.at[col_blk.at[slot, r + b + 2]], buf, sem)

                @pl.when(jnp.logical_and(r == B - 2, j + b + 2 < K))
                def _(buf=buf, sem=sem, b=b):
                    if b == 0:
                        pltpu.make_async_copy(
                            rows_hbm.at[c, s, 0], row_blk.at[slot2],
                            semr).wait()
                        pltpu.make_async_copy(
                            cols_hbm.at[c, s, 0], col_blk.at[slot2],
                            semc).wait()
                    pltpu.async_copy(
                        x_hbm.at[col_blk.at[slot2, b]], buf, sem)
                    if b == 1:
                        @pl.when(blk + 2 < NBLK)
                        def _():
                            pltpu.async_copy(
                                rows_hbm.at[c, s, blk + 2],
                                row_blk.at[slot], semr)
                            pltpu.async_copy(
                                cols_hbm.at[c, s, blk + 2],
                                col_blk.at[slot], semc)

        plsc.subcore_barrier()

        @pl.when(s < NS - 1)
        def _():
            pltpu.sync_copy(agg_sh.at[pl.ds(off, STRIPE)],
                            out_hbm.at[c, pl.ds(off, STRIPE)])

        @pl.when(s == NS - 1)
        def _():
            pltpu.sync_copy(
                agg_sh.at[pl.ds((NS - 1) * STRIPE, STRIPE_LAST)],
                out_hbm.at[c, pl.ds((NS - 1) * STRIPE, STRIPE_LAST)])

    return agg_kernel(x, rows_r, cols_r)


def _mlp_block(x_ref, a0_ref, a1_ref, w1_ref, b1_ref, w2_ref, b2_ref, o_ref):
    a = a0_ref[0] + a1_ref[0] - x_ref[...]
    h = jnp.dot(a, w1_ref[...], preferred_element_type=jnp.float32,
                precision=lax.Precision.HIGHEST) + b1_ref[...]
    h = jnp.maximum(h, 0.0)
    o_ref[...] = jnp.dot(h, w2_ref[...], preferred_element_type=jnp.float32,
                         precision=lax.Precision.HIGHEST) + b2_ref[...]


def _tc_mlp(x, agg, W1, b1, W2, b2):
    nb = N // ROW_BLK
    return pl.pallas_call(
        _mlp_block,
        grid=(nb,),
        in_specs=[
            pl.BlockSpec((ROW_BLK, D), lambda i: (i, 0)),
            pl.BlockSpec((1, ROW_BLK, D), lambda i: (0, i, 0)),
            pl.BlockSpec((1, ROW_BLK, D), lambda i: (1, i, 0)),
            pl.BlockSpec((D, D), lambda i: (0, 0)),
            pl.BlockSpec((1, D), lambda i: (0, 0)),
            pl.BlockSpec((D, D), lambda i: (0, 0)),
            pl.BlockSpec((1, D), lambda i: (0, 0)),
        ],
        out_specs=pl.BlockSpec((ROW_BLK, D), lambda i: (i, 0)),
        out_shape=jax.ShapeDtypeStruct((N, D), jnp.float32),
    )(x, agg, agg, W1, b1.reshape(1, D), W2, b2.reshape(1, D))


def kernel(x, edge_index, W1, b1, W2, b2):
    rows = edge_index[:, 0]
    cols = edge_index[:, 1]
    pad = E_PAD - E
    rows_p = jnp.concatenate(
        [rows, jnp.full((pad,), N, dtype=jnp.int32)]
    ).reshape(NC, NS, NBLK, B, CHUNK)
    cols_p = jnp.concatenate(
        [cols, jnp.zeros((pad,), dtype=jnp.int32)]
    ).reshape(NC, NS, NBLK, B, CHUNK)
    agg = _sc_aggregate(x, rows_p, cols_p)
    return _tc_mlp(x, agg, W1, b1, W2, b2)


# speedup vs baseline: 1.3205x; 1.3205x over previous
import functools

import jax
import jax.numpy as jnp
from jax import lax
from jax.experimental import pallas as pl
from jax.experimental.pallas import tpu as pltpu
from jax.experimental.pallas import tpu_sc as plsc

N = 10000
D = 128
E = 320000

NC = 2
NS = 16
NW = NC * NS
CHUNK = 128
PER_W = -(-E // (NW * CHUNK)) * CHUNK
K = PER_W // CHUNK
E_PAD = PER_W * NW
STRIPE = 632
STRIPE_LAST = N - STRIPE * (NS - 1)
N_PAD = N + 8

ROW_BLK = 1000


def _sc_aggregate(x, rows_r, cols_r):
    mesh = plsc.VectorSubcoreMesh(
        core_axis_name="c", subcore_axis_name="s",
        num_cores=NC, num_subcores=NS)

    @functools.partial(
        pl.kernel,
        out_type=jax.ShapeDtypeStruct((NC, N, D), jnp.float32),
        mesh=mesh,
        scratch_types=[
            pltpu.VMEM((K, CHUNK), jnp.int32),
            pltpu.VMEM((K, CHUNK), jnp.int32),
            pltpu.VMEM((CHUNK, D), jnp.float32),
            pltpu.VMEM_SHARED((N_PAD, D), jnp.float32),
            pltpu.SemaphoreType.DMA,
        ],
    )
    def agg_kernel(x_hbm, rows_hbm, cols_hbm, out_hbm,
                   row_v, col_v, buf, agg_sh, sem):
        c = lax.axis_index("c")
        s = lax.axis_index("s")
        pltpu.sync_copy(rows_hbm.at[c, s], row_v)
        pltpu.sync_copy(cols_hbm.at[c, s], col_v)
        off = pl.multiple_of(s * STRIPE, 8)

        @pl.when(s < NS - 1)
        def _():
            pltpu.sync_copy(x_hbm.at[pl.ds(off, STRIPE)],
                            agg_sh.at[pl.ds(off, STRIPE)])

        @pl.when(s == NS - 1)
        def _():
            pltpu.sync_copy(x_hbm.at[pl.ds((NS - 1) * STRIPE, STRIPE_LAST)],
                            agg_sh.at[pl.ds((NS - 1) * STRIPE, STRIPE_LAST)])

        plsc.subcore_barrier()

        H = CHUNK // 2

        @pl.loop(0, K)
        def _(j):
            h0 = pltpu.async_copy(
                x_hbm.at[col_v.at[j, pl.ds(0, H)]], buf.at[pl.ds(0, H)], sem)
            h1 = pltpu.async_copy(
                x_hbm.at[col_v.at[j, pl.ds(H, H)]], buf.at[pl.ds(H, H)], sem)
            h0.wait()
            h1.wait()
            pltpu.sync_copy(buf, agg_sh.at[row_v.at[j]], add=True)

        plsc.subcore_barrier()

        @pl.when(s < NS - 1)
        def _():
            pltpu.sync_copy(agg_sh.at[pl.ds(off, STRIPE)],
                            out_hbm.at[c, pl.ds(off, STRIPE)])

        @pl.when(s == NS - 1)
        def _():
            pltpu.sync_copy(
                agg_sh.at[pl.ds((NS - 1) * STRIPE, STRIPE_LAST)],
                out_hbm.at[c, pl.ds((NS - 1) * STRIPE, STRIPE_LAST)])

    return agg_kernel(x, rows_r, cols_r)


def _mlp_block(x_ref, a0_ref, a1_ref, w1_ref, b1_ref, w2_ref, b2_ref, o_ref):
    a = a0_ref[0] + a1_ref[0] - x_ref[...]
    h = jnp.dot(a, w1_ref[...], preferred_element_type=jnp.float32,
                precision=lax.Precision.HIGHEST) + b1_ref[...]
    h = jnp.maximum(h, 0.0)
    o_ref[...] = jnp.dot(h, w2_ref[...], preferred_element_type=jnp.float32,
                         precision=lax.Precision.HIGHEST) + b2_ref[...]


def _tc_mlp(x, agg, W1, b1, W2, b2):
    nb = N // ROW_BLK
    return pl.pallas_call(
        _mlp_block,
        grid=(nb,),
        in_specs=[
            pl.BlockSpec((ROW_BLK, D), lambda i: (i, 0)),
            pl.BlockSpec((1, ROW_BLK, D), lambda i: (0, i, 0)),
            pl.BlockSpec((1, ROW_BLK, D), lambda i: (1, i, 0)),
            pl.BlockSpec((D, D), lambda i: (0, 0)),
            pl.BlockSpec((1, D), lambda i: (0, 0)),
            pl.BlockSpec((D, D), lambda i: (0, 0)),
            pl.BlockSpec((1, D), lambda i: (0, 0)),
        ],
        out_specs=pl.BlockSpec((ROW_BLK, D), lambda i: (i, 0)),
        out_shape=jax.ShapeDtypeStruct((N, D), jnp.float32),
    )(x, agg, agg, W1, b1.reshape(1, D), W2, b2.reshape(1, D))


def kernel(x, edge_index, W1, b1, W2, b2):
    rows = edge_index[:, 0]
    cols = edge_index[:, 1]
    pad = E_PAD - E
    rows_p = jnp.concatenate(
        [rows, jnp.full((pad,), N, dtype=jnp.int32)]
    ).reshape(NC, NS, K, CHUNK)
    cols_p = jnp.concatenate(
        [cols, jnp.zeros((pad,), dtype=jnp.int32)]
    ).reshape(NC, NS, K, CHUNK)
    agg = _sc_aggregate(x, rows_p, cols_p)
    return _tc_mlp(x, agg, W1, b1, W2, b2)
